# baseline (speedup 1.0000x reference)
import functools

import jax
import jax.numpy as jnp
from jax import lax
from jax.experimental import pallas as pl
from jax.experimental.pallas import tpu as pltpu
from jax.experimental.pallas import tpu_sc as plsc

N = 10000
E = 320000
D = 128

NC = 2
NS = 16
NW = NC * NS
CH = 128
NCH = 80
EPAD = NW * NCH * CH
NDUM = 10240
RPT = NDUM // NS

_mesh = plsc.VectorSubcoreMesh(core_axis_name="c", subcore_axis_name="s")


@functools.partial(
    pl.kernel,
    out_type=jax.ShapeDtypeStruct((NC, NDUM), jnp.float32),
    mesh=_mesh,
    scratch_types=[
        pltpu.VMEM((NCH, CH), jnp.int32),
        pltpu.VMEM((CH,), jnp.float32),
        pltpu.VMEM((RPT,), jnp.float32),
        pltpu.VMEM_SHARED((NDUM,), jnp.float32),
    ],
)
def _deg_kernel(dst_hbm, degp_hbm, idx_v, ones_v, z_v, acc_s):
    c = lax.axis_index("c")
    s = lax.axis_index("s")
    wid = c * NS + s

    one16 = jnp.ones((16,), jnp.float32)
    zero16 = jnp.zeros((16,), jnp.float32)
    for k in range(CH // 16):
        ones_v[pl.ds(k * 16, 16)] = one16

    def _zb(r, _):
        z_v[pl.ds(r * 16, 16)] = zero16
        return 0
    lax.fori_loop(0, RPT // 16, _zb, 0)
    pltpu.sync_copy(z_v, acc_s.at[pl.ds(s * RPT, RPT)])
    pltpu.sync_copy(dst_hbm.at[wid], idx_v)
    plsc.subcore_barrier()

    def _body(j, _):
        pltpu.sync_copy(ones_v, acc_s.at[idx_v.at[j]], add=True)
        return 0
    lax.fori_loop(0, NCH, _body, 0)
    plsc.subcore_barrier()

    pltpu.sync_copy(acc_s.at[pl.ds(s * RPT, RPT)],
                    degp_hbm.at[c, pl.ds(s * RPT, RPT)])


NBUF = 2
NISL = 4
NCH0 = 156
NCH1 = 2 * NCH - NCH0
TOTCH = NW * NCH


@functools.partial(
    pl.kernel,
    out_type=jax.ShapeDtypeStruct((NC, NDUM, D), jnp.float32),
    mesh=_mesh,
    scratch_types=[
        *([pltpu.VMEM((2, CH), jnp.int32)] * NISL),
        *([pltpu.VMEM((CH, D), jnp.float32)] * NBUF),
        pltpu.VMEM_SHARED((NDUM, D), jnp.float32),
        *([pltpu.SemaphoreType.DMA] * NISL),
        *([pltpu.SemaphoreType.DMA] * NBUF),
        *([pltpu.SemaphoreType.DMA] * NBUF),
    ],
)
def _agg_kernel(xs_hbm, idx_hbm, parts_hbm,
                i0, i1, i2, i3, bufa, bufb, acc_s,
                ia, ib, ic, id_, ga, gb, sa, sb):
    isl = (i0, i1, i2, i3)
    isem = (ia, ib, ic, id_)
    bufs = (bufa, bufb)
    gsem = (ga, gb)
    ssem = (sa, sb)
    c = lax.axis_index("c")
    s = lax.axis_index("s")

    def _fire_i(chunk, m):
        pltpu.async_copy(idx_hbm.at[chunk], isl[m], isem[m])

    def _wait_i(m):
        pltpu.make_async_copy(idx_hbm.at[0], isl[m], isem[m]).wait()

    def _fire_g(m, b):
        pltpu.async_copy(xs_hbm.at[isl[m].at[0]], bufs[b], gsem[b])

    def _wait_g(b):
        pltpu.make_async_copy(xs_hbm.at[isl[0].at[0]], bufs[b], gsem[b]).wait()

    def _fire_s(m, b):
        pltpu.async_copy(bufs[b], acc_s.at[isl[m].at[1]], ssem[b],
                         add=True)

    def _wait_s(b):
        pltpu.make_async_copy(bufs[b], acc_s.at[isl[0].at[1]], ssem[b]).wait()

    zero16 = jnp.zeros((16,), jnp.float32)

    def _zb(r, _):
        for k in range(D // 16):
            bufs[0][r, pl.ds(k * 16, 16)] = zero16
        return 0
    lax.fori_loop(0, CH, _zb, 0)
    for k in range(RPT // CH):
        pltpu.sync_copy(bufs[0], acc_s.at[pl.ds(s * RPT + k * CH, CH)])
    plsc.subcore_barrier()

    def _pipe(nch, base, fire_g, wait_g):
        _fire_i(base + 0, 0)
        _fire_i(base + 1, 1)
        _wait_i(0)
        fire_g(0, 0)
        wait_g(0)
        _fire_s(0, 0)
        _fire_i(base + 2, 2)
        _wait_i(1)
        fire_g(1, 1)
        wait_g(1)
        _fire_s(1, 1)
        _wait_s(0)
        _fire_i(base + 3, 3)
        _wait_i(2)
        fire_g(2, 0)

        def _outer(t, _):
            j0 = base + 2 + t * NISL
            for q in range(NISL):
                j = j0 + q
                b = q % NBUF
                wait_g(b)
                _fire_s((2 + q) % NISL, b)
                _wait_s(1 - b)
                _fire_i(j + 2, q % NISL)
                _wait_i((3 + q) % NISL)
                fire_g((3 + q) % NISL, 1 - b)
            return 0
        lax.fori_loop(0, (nch - 4) // NISL, _outer, 0)

        wait_g(0)
        _fire_s((nch - 2) % NISL, 0)
        _wait_s(1)
        _wait_i((nch - 1) % NISL)
        fire_g((nch - 1) % NISL, 1)
        wait_g(1)
        _fire_s((nch - 1) % NISL, 1)
        _wait_s(0)
        _wait_s(1)

    @pl.when(c == 0)
    def _():
        _pipe(NCH0, s * NCH0, _fire_g, _wait_g)

    if NCH1 > 0:
        @pl.when(c == 1)
        def _():
            _pipe(NCH1, NS * NCH0 + s * NCH1, _fire_g, _wait_g)

    plsc.subcore_barrier()

    for k in range(RPT // CH):
        pltpu.sync_copy(acc_s.at[pl.ds(s * RPT + k * CH, CH)],
                        parts_hbm.at[c, pl.ds(s * RPT + k * CH, CH)])


def _prescale_body(degp_ref, x_ref, xs_ref, dis_ref):
    deg = degp_ref[0, :] + degp_ref[1, :] + 1.0
    dis = lax.rsqrt(deg)
    dis2 = dis[:, None]
    dis_ref[...] = dis2
    xs_ref[...] = x_ref[...] * dis2


_prescale_call = pl.pallas_call(
    _prescale_body,
    out_shape=(
        jax.ShapeDtypeStruct((NDUM, D), jnp.float32),
        jax.ShapeDtypeStruct((NDUM, 1), jnp.float32),
    ),
)


def _final_body(parts_ref, xs_ref, dis_ref, w_ref, b_ref, out_ref):
    srow = (parts_ref[0] + parts_ref[1] + xs_ref[...]) * dis_ref[...]
    out_ref[...] = (
        jnp.dot(srow, w_ref[...], preferred_element_type=jnp.float32)
        + b_ref[...]
    )


_BR = 1000

_final_call = pl.pallas_call(
    _final_body,
    grid=(N // _BR,),
    in_specs=[
        pl.BlockSpec((NC, _BR, D), lambda i: (0, i, 0)),
        pl.BlockSpec((_BR, D), lambda i: (i, 0)),
        pl.BlockSpec((_BR, 1), lambda i: (i, 0)),
        pl.BlockSpec((D, D), lambda i: (0, 0)),
        pl.BlockSpec((1, D), lambda i: (0, 0)),
    ],
    out_specs=pl.BlockSpec((_BR, D), lambda i: (i, 0)),
    out_shape=jax.ShapeDtypeStruct((N, D), jnp.float32),
)


def kernel(x, edge_index, W, b):
    src = edge_index[0]
    dst = edge_index[1]
    pad = EPAD - E
    src3 = jnp.concatenate([src, jnp.zeros((pad,), jnp.int32)]).reshape(NW, NCH, CH)
    dst3 = jnp.concatenate([dst, jnp.full((pad,), N, jnp.int32)]).reshape(NW, NCH, CH)
    x_pad = jnp.pad(x, ((0, NDUM - N), (0, 0)))

    idxc = jnp.stack([src3, dst3], axis=2).reshape(TOTCH, 2, CH)

    degp = _deg_kernel(dst3)
    xs, dis = _prescale_call(degp, x_pad)
    parts = _agg_kernel(xs, idxc)
    out = _final_call(parts, xs, dis, W, b.reshape(1, D))
    return out


# speedup vs baseline: 1.1177x; 1.1177x over previous
import functools

import jax
import jax.numpy as jnp
from jax import lax
from jax.experimental import pallas as pl
from jax.experimental.pallas import tpu as pltpu
from jax.experimental.pallas import tpu_sc as plsc

N = 10000
E = 320000
D = 128

NC = 2
NS = 16
NW = NC * NS
CH = 128
NCH = 80
EPAD = NW * NCH * CH
NDUM = 10240
RPT = NDUM // NS

_mesh = plsc.VectorSubcoreMesh(core_axis_name="c", subcore_axis_name="s")


@functools.partial(
    pl.kernel,
    out_type=jax.ShapeDtypeStruct((NC, NDUM), jnp.float32),
    mesh=_mesh,
    scratch_types=[
        pltpu.VMEM((NCH, CH), jnp.int32),
        pltpu.VMEM((CH,), jnp.float32),
        pltpu.VMEM((RPT,), jnp.float32),
        pltpu.VMEM_SHARED((NDUM,), jnp.float32),
    ],
)
def _deg_kernel(dst_hbm, degp_hbm, idx_v, ones_v, z_v, acc_s):
    c = lax.axis_index("c")
    s = lax.axis_index("s")
    wid = c * NS + s

    one16 = jnp.ones((16,), jnp.float32)
    zero16 = jnp.zeros((16,), jnp.float32)
    for k in range(CH // 16):
        ones_v[pl.ds(k * 16, 16)] = one16

    def _zb(r, _):
        z_v[pl.ds(r * 16, 16)] = zero16
        return 0
    lax.fori_loop(0, RPT // 16, _zb, 0)
    pltpu.sync_copy(z_v, acc_s.at[pl.ds(s * RPT, RPT)])
    pltpu.sync_copy(dst_hbm.at[wid], idx_v)
    plsc.subcore_barrier()

    def _body(j, _):
        pltpu.sync_copy(ones_v, acc_s.at[idx_v.at[j]], add=True)
        return 0
    lax.fori_loop(0, NCH, _body, 0)
    plsc.subcore_barrier()

    pltpu.sync_copy(acc_s.at[pl.ds(s * RPT, RPT)],
                    degp_hbm.at[c, pl.ds(s * RPT, RPT)])


NBUF = 2
NISL = 4
NCH0 = 152
NCH1 = 2 * NCH - NCH0
TOTCH = NW * NCH


@functools.partial(
    pl.kernel,
    out_type=jax.ShapeDtypeStruct((NC, NDUM, D), jnp.float32),
    mesh=_mesh,
    scratch_types=[
        *([pltpu.VMEM((2, CH), jnp.int32)] * NISL),
        *([pltpu.VMEM((CH, D), jnp.float32)] * NBUF),
        pltpu.VMEM_SHARED((NDUM, D), jnp.float32),
        *([pltpu.SemaphoreType.DMA] * NISL),
        *([pltpu.SemaphoreType.DMA] * NBUF),
        *([pltpu.SemaphoreType.DMA] * NBUF),
    ],
)
def _agg_kernel(xs_hbm, idx_hbm, parts_hbm,
                i0, i1, i2, i3, bufa, bufb, acc_s,
                ia, ib, ic, id_, ga, gb, sa, sb):
    isl = (i0, i1, i2, i3)
    isem = (ia, ib, ic, id_)
    bufs = (bufa, bufb)
    gsem = (ga, gb)
    ssem = (sa, sb)
    c = lax.axis_index("c")
    s = lax.axis_index("s")

    def _fire_i(chunk, m):
        pltpu.async_copy(idx_hbm.at[chunk], isl[m], isem[m])

    def _wait_i(m):
        pltpu.make_async_copy(idx_hbm.at[0], isl[m], isem[m]).wait()

    def _fire_g(m, b):
        pltpu.async_copy(xs_hbm.at[isl[m].at[0]], bufs[b], gsem[b])

    def _wait_g(b):
        pltpu.make_async_copy(xs_hbm.at[isl[0].at[0]], bufs[b], gsem[b]).wait()

    def _fire_s(m, b):
        pltpu.async_copy(bufs[b], acc_s.at[isl[m].at[1]], ssem[b],
                         add=True)

    def _wait_s(b):
        pltpu.make_async_copy(bufs[b], acc_s.at[isl[0].at[1]], ssem[b]).wait()

    zero16 = jnp.zeros((16,), jnp.float32)

    def _zb(r, _):
        for k in range(D // 16):
            bufs[0][r, pl.ds(k * 16, 16)] = zero16
        return 0
    lax.fori_loop(0, CH, _zb, 0)
    for k in range(RPT // CH):
        pltpu.sync_copy(bufs[0], acc_s.at[pl.ds(s * RPT + k * CH, CH)])
    plsc.subcore_barrier()

    def _pipe(nch, base, fire_g, wait_g):
        _fire_i(base + 0, 0)
        _fire_i(base + 1, 1)
        _wait_i(0)
        fire_g(0, 0)
        wait_g(0)
        _fire_s(0, 0)
        _fire_i(base + 2, 2)
        _wait_i(1)
        fire_g(1, 1)
        wait_g(1)
        _fire_s(1, 1)
        _wait_s(0)
        _fire_i(base + 3, 3)
        _wait_i(2)
        fire_g(2, 0)

        def _outer(t, _):
            j0 = base + 2 + t * NISL
            for q in range(NISL):
                j = j0 + q
                b = q % NBUF
                wait_g(b)
                _fire_s((2 + q) % NISL, b)
                _wait_s(1 - b)
                _fire_i(j + 2, q % NISL)
                _wait_i((3 + q) % NISL)
                fire_g((3 + q) % NISL, 1 - b)
            return 0
        lax.fori_loop(0, (nch - 4) // NISL, _outer, 0)

        wait_g(0)
        _fire_s((nch - 2) % NISL, 0)
        _wait_s(1)
        _wait_i((nch - 1) % NISL)
        fire_g((nch - 1) % NISL, 1)
        wait_g(1)
        _fire_s((nch - 1) % NISL, 1)
        _wait_s(0)
        _wait_s(1)

    @pl.when(c == 0)
    def _():
        _pipe(NCH0, s * NCH0, _fire_g, _wait_g)

    if NCH1 > 0:
        @pl.when(c == 1)
        def _():
            _pipe(NCH1, NS * NCH0 + s * NCH1, _fire_g, _wait_g)

    plsc.subcore_barrier()

    for k in range(RPT // CH):
        pltpu.sync_copy(acc_s.at[pl.ds(s * RPT + k * CH, CH)],
                        parts_hbm.at[c, pl.ds(s * RPT + k * CH, CH)])


def _prescale_body(degp_ref, x_ref, xs_ref, dis_ref):
    deg = degp_ref[0, :] + degp_ref[1, :] + 1.0
    dis = lax.rsqrt(deg)
    dis2 = dis[:, None]
    dis_ref[...] = dis2
    xs_ref[...] = x_ref[...] * dis2


_prescale_call = pl.pallas_call(
    _prescale_body,
    out_shape=(
        jax.ShapeDtypeStruct((NDUM, D), jnp.float32),
        jax.ShapeDtypeStruct((NDUM, 1), jnp.float32),
    ),
)


def _final_body(parts_ref, xs_ref, dis_ref, w_ref, b_ref, out_ref):
    srow = (parts_ref[0] + parts_ref[1] + xs_ref[...]) * dis_ref[...]
    out_ref[...] = (
        jnp.dot(srow, w_ref[...], preferred_element_type=jnp.float32)
        + b_ref[...]
    )


_BR = 1000

_final_call = pl.pallas_call(
    _final_body,
    grid=(N // _BR,),
    in_specs=[
        pl.BlockSpec((NC, _BR, D), lambda i: (0, i, 0)),
        pl.BlockSpec((_BR, D), lambda i: (i, 0)),
        pl.BlockSpec((_BR, 1), lambda i: (i, 0)),
        pl.BlockSpec((D, D), lambda i: (0, 0)),
        pl.BlockSpec((1, D), lambda i: (0, 0)),
    ],
    out_specs=pl.BlockSpec((_BR, D), lambda i: (i, 0)),
    out_shape=jax.ShapeDtypeStruct((N, D), jnp.float32),
)


def kernel(x, edge_index, W, b):
    src = edge_index[0]
    dst = edge_index[1]
    pad = EPAD - E
    src3 = jnp.concatenate([src, jnp.zeros((pad,), jnp.int32)]).reshape(NW, NCH, CH)
    dst3 = jnp.concatenate([dst, jnp.full((pad,), N, jnp.int32)]).reshape(NW, NCH, CH)
    x_pad = jnp.pad(x, ((0, NDUM - N), (0, 0)))

    idxc = jnp.stack([src3, dst3], axis=2).reshape(TOTCH, 2, CH)

    degp = _deg_kernel(dst3)
    xs, dis = _prescale_call(degp, x_pad)
    parts = _agg_kernel(xs, idxc)
    out = _final_call(parts, xs, dis, W, b.reshape(1, D))
    return out
